# baseline (speedup 1.0000x reference)
import functools

import jax
import jax.numpy as jnp
from jax.experimental import pallas as pl
from jax.experimental.pallas import tpu as pltpu

B, E, D, P = 16384, 64, 64, 4096
S, L = 32, 128
R = 128
NB = B // R


def _encw_body(enc2_ref, w2_ref, out_ref):
    out_ref[...] = jnp.dot(enc2_ref[...], w2_ref[...],
                           preferred_element_type=jnp.float32)


def _body(sidx_ref, order_ref, starts_ref,
          h_ref, k_ref, enc2_ref, encw2_ref, u2_ref, v2_ref, ones2_ref,
          out_ref,
          acc_ref, hk_ref, hukv_ref):
    i = pl.program_id(0)

    mlow = jax.lax.broadcasted_iota(jnp.int32, (S, L), 1) < (L // 2)

    hb = h_ref[...]
    kb = k_ref[...]
    hp = jnp.concatenate([hb[:, :S, :], hb[:, S:, :]], axis=2)
    kp = jnp.concatenate([kb[:, :S, :], kb[:, S:, :]], axis=2)

    acc_ref[...] = hp

    h2 = hp.reshape(R * S, L)
    k2 = kp.reshape(R * S, L)
    hukv_ref[...] = (
        jnp.dot(h2, u2_ref[...], preferred_element_type=jnp.float32)
        + jnp.dot(k2, v2_ref[...], preferred_element_type=jnp.float32)
    ).reshape(R, S, L)
    hk_ref[...] = hp + kp

    s = starts_ref[i]
    e = starts_ref[i + 1]

    UNROLL = 4

    def step(t, _):
        jj = s + t * UNROLL
        for u_off in range(UNROLL):
            j = jnp.minimum(jj + u_off, e - 1)
            valid = (jj + u_off) < e
            r = sidx_ref[j] - i * R
            o = order_ref[j]
            prod = hk_ref[r] * enc2_ref[o][None, :]
            glo = jnp.sum(jnp.where(mlow, prod, 0.0), axis=1, keepdims=True)
            ghi = jnp.sum(jnp.where(mlow, 0.0, prod), axis=1, keepdims=True)
            gate = jax.nn.sigmoid(jnp.where(mlow, glo, ghi))
            gate = jnp.where(valid, gate, 0.0)
            ht = jax.nn.relu(hukv_ref[r] + encw2_ref[o][None, :])
            acc_ref[r] = acc_ref[r] + gate * ht
        return _

    jax.lax.fori_loop(0, (e - s + UNROLL - 1) // UNROLL, step, None)

    x = acc_ref[...]
    x2 = x.reshape(R * S, L)
    ssel = jnp.dot(x2 * x2, ones2_ref[...],
                   preferred_element_type=jnp.float32).reshape(R, S, L)
    y = x * jax.lax.rsqrt(jnp.maximum(ssel, 1e-12))
    out_ref[...] = jnp.concatenate([y[:, :, :D], y[:, :, D:]], axis=1)


@functools.partial(jax.jit, static_argnames=("interpret",))
def kernel(hiddens, keys, encoded_sents, indices, U, V, W, interpret=False):
    order = jnp.argsort(indices).astype(jnp.int32)
    sidx = indices[order].astype(jnp.int32)
    starts = jnp.searchsorted(
        sidx, (jnp.arange(NB + 1) * R).astype(jnp.int32), side="left"
    ).astype(jnp.int32)

    enc2 = jnp.concatenate([encoded_sents, encoded_sents], axis=1)
    z = jnp.zeros((D, D), jnp.float32)
    u2 = jnp.block([[U, z], [z, U]])
    v2 = jnp.block([[V, z], [z, V]])
    w2 = jnp.block([[W, z], [z, W]])
    o1 = jnp.ones((D, D), jnp.float32)
    ones2 = jnp.block([[o1, z], [z, o1]])

    encw2 = pl.pallas_call(
        _encw_body,
        out_shape=jax.ShapeDtypeStruct((P, L), jnp.float32),
        interpret=interpret,
    )(enc2, w2)

    grid_spec = pltpu.PrefetchScalarGridSpec(
        num_scalar_prefetch=3,
        grid=(NB,),
        in_specs=[
            pl.BlockSpec((R, E, D), lambda i, *_: (i, 0, 0)),
            pl.BlockSpec((R, E, D), lambda i, *_: (i, 0, 0)),
            pl.BlockSpec((P, L), lambda i, *_: (0, 0)),
            pl.BlockSpec((P, L), lambda i, *_: (0, 0)),
            pl.BlockSpec((L, L), lambda i, *_: (0, 0)),
            pl.BlockSpec((L, L), lambda i, *_: (0, 0)),
            pl.BlockSpec((L, L), lambda i, *_: (0, 0)),
        ],
        out_specs=pl.BlockSpec((R, E, D), lambda i, *_: (i, 0, 0)),
        scratch_shapes=[pltpu.VMEM((R, S, L), jnp.float32),
                        pltpu.VMEM((R, S, L), jnp.float32),
                        pltpu.VMEM((R, S, L), jnp.float32)],
    )
    return pl.pallas_call(
        _body,
        grid_spec=grid_spec,
        out_shape=jax.ShapeDtypeStruct((B, E, D), jnp.float32),
        compiler_params=pltpu.CompilerParams(
            dimension_semantics=("parallel",)),
        interpret=interpret,
    )(sidx, order, starts, hiddens, keys, enc2, encw2, u2, v2, ones2)


# speedup vs baseline: 1.7010x; 1.7010x over previous
import functools

import jax
import jax.numpy as jnp
from jax.experimental import pallas as pl
from jax.experimental.pallas import tpu as pltpu

B, E, D, P = 16384, 64, 64, 4096
S, L = 32, 128
R = 256
NB = B // R


def _encw_body(enc2_ref, w2_ref, out_ref):
    out_ref[...] = jnp.dot(enc2_ref[...], w2_ref[...],
                           preferred_element_type=jnp.float32)


def _body(sidx_ref, order_ref, starts_ref,
          h_ref, k_ref, enc2_ref, encw2_ref, u2_ref, v2_ref, ones2_ref,
          out_ref,
          acc_ref, hk_ref, hukv_ref):
    i = pl.program_id(0)

    mlow = jax.lax.broadcasted_iota(jnp.int32, (S, L), 1) < (L // 2)

    hp = h_ref[...]
    kp = k_ref[...]
    acc_ref[...] = hp

    h2 = hp.reshape(R * S, L)
    k2 = kp.reshape(R * S, L)
    hukv_ref[...] = (
        jnp.dot(h2, u2_ref[...], preferred_element_type=jnp.float32)
        + jnp.dot(k2, v2_ref[...], preferred_element_type=jnp.float32)
    ).reshape(R, S, L)
    hk_ref[...] = hp + kp

    s = starts_ref[i]
    e = starts_ref[i + 1]

    UNROLL = 4

    def step(t, _):
        jj = s + t * UNROLL
        for u_off in range(UNROLL):
            j = jnp.minimum(jj + u_off, e - 1)
            valid = (jj + u_off) < e
            r = sidx_ref[j] - i * R
            o = order_ref[j]
            prod = hk_ref[r] * enc2_ref[o][None, :]
            glo = jnp.sum(jnp.where(mlow, prod, 0.0), axis=1, keepdims=True)
            ghi = jnp.sum(jnp.where(mlow, 0.0, prod), axis=1, keepdims=True)
            gate = jax.nn.sigmoid(jnp.where(mlow, glo, ghi))
            gate = jnp.where(valid, gate, 0.0)
            ht = jax.nn.relu(hukv_ref[r] + encw2_ref[o][None, :])
            acc_ref[r] = acc_ref[r] + gate * ht
        return _

    jax.lax.fori_loop(0, (e - s + UNROLL - 1) // UNROLL, step, None)

    x = acc_ref[...]
    x2 = x.reshape(R * S, L)
    ssel = jnp.dot(x2 * x2, ones2_ref[...],
                   preferred_element_type=jnp.float32).reshape(R, S, L)
    out_ref[...] = x * jax.lax.rsqrt(jnp.maximum(ssel, 1e-12))


@functools.partial(jax.jit, static_argnames=("interpret",))
def kernel(hiddens, keys, encoded_sents, indices, U, V, W, interpret=False):
    order = jnp.argsort(indices).astype(jnp.int32)
    sidx = indices[order].astype(jnp.int32)
    starts = jnp.searchsorted(
        sidx, (jnp.arange(NB + 1) * R).astype(jnp.int32), side="left"
    ).astype(jnp.int32)

    h3 = hiddens.reshape(B, S, L)
    k3 = keys.reshape(B, S, L)
    enc2 = jnp.concatenate([encoded_sents, encoded_sents], axis=1)
    z = jnp.zeros((D, D), jnp.float32)
    u2 = jnp.block([[U, z], [z, U]])
    v2 = jnp.block([[V, z], [z, V]])
    w2 = jnp.block([[W, z], [z, W]])
    o1 = jnp.ones((D, D), jnp.float32)
    ones2 = jnp.block([[o1, z], [z, o1]])

    encw2 = pl.pallas_call(
        _encw_body,
        out_shape=jax.ShapeDtypeStruct((P, L), jnp.float32),
        interpret=interpret,
    )(enc2, w2)

    grid_spec = pltpu.PrefetchScalarGridSpec(
        num_scalar_prefetch=3,
        grid=(NB,),
        in_specs=[
            pl.BlockSpec((R, S, L), lambda i, *_: (i, 0, 0)),
            pl.BlockSpec((R, S, L), lambda i, *_: (i, 0, 0)),
            pl.BlockSpec((P, L), lambda i, *_: (0, 0)),
            pl.BlockSpec((P, L), lambda i, *_: (0, 0)),
            pl.BlockSpec((L, L), lambda i, *_: (0, 0)),
            pl.BlockSpec((L, L), lambda i, *_: (0, 0)),
            pl.BlockSpec((L, L), lambda i, *_: (0, 0)),
        ],
        out_specs=pl.BlockSpec((R, S, L), lambda i, *_: (i, 0, 0)),
        scratch_shapes=[pltpu.VMEM((R, S, L), jnp.float32),
                        pltpu.VMEM((R, S, L), jnp.float32),
                        pltpu.VMEM((R, S, L), jnp.float32)],
    )
    out = pl.pallas_call(
        _body,
        grid_spec=grid_spec,
        out_shape=jax.ShapeDtypeStruct((B, S, L), jnp.float32),
        compiler_params=pltpu.CompilerParams(
            dimension_semantics=("parallel",)),
        interpret=interpret,
    )(sidx, order, starts, h3, k3, enc2, encw2, u2, v2, ones2)
    return out.reshape(B, E, D)
